# Initial kernel scaffold; baseline (speedup 1.0000x reference)
import jax
import jax.numpy as jnp
from jax.experimental import pallas as pl


def kernel(z, codebook):
    raise NotImplementedError("write your pallas kernel here")


# speedup vs baseline: 2.2714x; 2.2714x over previous
import jax
import jax.numpy as jnp
from jax import lax
from jax.experimental import pallas as pl


_B, _D, _T, _K = 32, 64, 1024, 1024


def _vq_body(z_ref, cb_ref, zq_ref, idx_ref, sse_ref):
    b = pl.program_id(0)
    zb = z_ref[0]
    cb = cb_ref[...]

    p = lax.dot_general(zb, cb, (((0,), (1,)), ((), ())),
                        preferred_element_type=jnp.float32)
    rn = jnp.sum(zb * zb, axis=0)
    cn = jnp.sum(cb * cb, axis=1)
    dist = rn[:, None] - 2.0 * p + cn[None, :]

    m = jnp.min(dist, axis=1, keepdims=True)
    kk = lax.broadcasted_iota(jnp.int32, (_T, _K), 1)
    idx = jnp.min(jnp.where(dist == m, kk, _K), axis=1)
    idx_ref[0, 0, :] = idx

    oh = (lax.broadcasted_iota(jnp.int32, (_K, _T), 0)
          == idx[None, :]).astype(jnp.float32)
    zq = lax.dot_general(cb, oh, (((0,), (0,)), ((), ())),
                         preferred_element_type=jnp.float32)

    zq_ref[0] = zb + (zq - zb)

    r = zb - zq
    part = jnp.sum(r * r).reshape(1, 1)

    @pl.when(b == 0)
    def _init():
        sse_ref[...] = part

    @pl.when(b != 0)
    def _acc():
        sse_ref[...] = sse_ref[...] + part


def kernel(z, codebook):
    zq, idx3, sse = pl.pallas_call(
        _vq_body,
        grid=(_B,),
        in_specs=[
            pl.BlockSpec((1, _D, _T), lambda b: (b, 0, 0)),
            pl.BlockSpec((_K, _D), lambda b: (0, 0)),
        ],
        out_specs=[
            pl.BlockSpec((1, _D, _T), lambda b: (b, 0, 0)),
            pl.BlockSpec((1, 1, _T), lambda b: (b, 0, 0)),
            pl.BlockSpec((1, 1), lambda b: (0, 0)),
        ],
        out_shape=[
            jax.ShapeDtypeStruct((_B, _D, _T), jnp.float32),
            jax.ShapeDtypeStruct((_B, 1, _T), jnp.int32),
            jax.ShapeDtypeStruct((1, 1), jnp.float32),
        ],
    )(z, codebook)
    mean_sq = sse[0, 0] / (_B * _T * _D)
    vq_loss = mean_sq + 0.25 * mean_sq
    return zq, idx3.reshape(_B, _T), vq_loss
